# baseline (speedup 1.0000x reference)
import functools

import jax
import jax.numpy as jnp
from jax import lax
from jax.experimental import pallas as pl
from jax.experimental.pallas import tpu as pltpu
from jax.experimental.pallas import tpu_sc as plsc

NUM_BIOMES = 11
EMBED_DIM = 64
BATCH = 16384
_PAD_R = 16
_PAD_D = 128

_info = plsc.get_sparse_core_info()
_NC, _NS = _info.num_cores, _info.num_subcores
_NW = _NC * _NS
_B_PER_W = BATCH // _NW
_CHUNK = 128
_N_CHUNK = _B_PER_W // _CHUNK
_L = 16
_ROWS_PER_STEP = 8


def _make_gather():
    mesh = plsc.VectorSubcoreMesh(core_axis_name="c", subcore_axis_name="s")

    @functools.partial(
        pl.kernel,
        mesh=mesh,
        out_type=jax.ShapeDtypeStruct((BATCH, EMBED_DIM), jnp.float32),
        compiler_params=pltpu.CompilerParams(
            skip_device_barrier=True,
            disable_semaphore_checks=True,
        ),
        scratch_types=[
            pltpu.VMEM((_N_CHUNK, _CHUNK), jnp.int32),
            pltpu.VMEM((2, _CHUNK, _PAD_D), jnp.float32),
            pltpu.VMEM((2, _CHUNK, EMBED_DIM), jnp.float32),
            pltpu.SemaphoreType.DMA,
            pltpu.SemaphoreType.DMA,
        ],
    )
    def gather_kernel(idx_hbm, table_hbm, out_hbm, idx_v, wide_v, rows_v,
                      sem_g, sem_w):
        sid = lax.axis_index("s")
        wid = sid * _NC + lax.axis_index("c")
        base = wid * _B_PER_W
        for j in range(_N_CHUNK):
            pltpu.sync_copy(idx_hbm.at[pl.ds(base + j * _CHUNK, _CHUNK)],
                            idx_v.at[j])

        def compact(j):
            def body(step, carry):
                r0 = step * _ROWS_PER_STEP
                for r in range(_ROWS_PER_STEP):
                    for c in range(EMBED_DIM // _L):
                        rows_v[j % 2, r0 + r, pl.ds(c * _L, _L)] = (
                            wide_v[j % 2, r0 + r, pl.ds(c * _L, _L)])
                return carry
            lax.fori_loop(0, _CHUNK // _ROWS_PER_STEP, body, 0)

        def fire_gather(j):
            return pltpu.async_copy(
                table_hbm.at[idx_v.at[j]], wide_v.at[j % 2], sem_g)

        gathers = [fire_gather(0)]
        writes = []
        for j in range(_N_CHUNK):
            gathers[j].wait()
            if j + 1 < _N_CHUNK:
                gathers.append(fire_gather(j + 1))
            if j >= 2:
                writes[j - 2].wait()
            compact(j)
            writes.append(pltpu.async_copy(
                rows_v.at[j % 2],
                out_hbm.at[pl.ds(base + j * _CHUNK, _CHUNK)],
                sem_w,
            ))
        for w in writes[max(0, _N_CHUNK - 2):]:
            w.wait()

    return gather_kernel


_gather = _make_gather()


def kernel(biome_labels, table):
    idx = biome_labels.astype(jnp.int32)
    padded = jnp.pad(
        table, ((0, _PAD_R - NUM_BIOMES), (0, _PAD_D - EMBED_DIM)))
    return _gather(idx, padded)


# speedup vs baseline: 1.7410x; 1.7410x over previous
import functools

import jax
import jax.numpy as jnp
from jax import lax
from jax.experimental import pallas as pl
from jax.experimental.pallas import tpu as pltpu
from jax.experimental.pallas import tpu_sc as plsc

NUM_BIOMES = 11
EMBED_DIM = 64
BATCH = 16384
_ROWS = 12
_PAIR_D = 2 * EMBED_DIM
_HALF = BATCH // 2

_info = plsc.get_sparse_core_info()
_NC, _NS = _info.num_cores, _info.num_subcores
_NW = _NC * _NS
_P_PER_W = _HALF // _NW
_CHUNK = 128
_N_CHUNK = _P_PER_W // _CHUNK
_L = 16


def _make_gather():
    mesh = plsc.VectorSubcoreMesh(core_axis_name="c", subcore_axis_name="s")

    @functools.partial(
        pl.kernel,
        mesh=mesh,
        out_type=jax.ShapeDtypeStruct((_HALF, _PAIR_D), jnp.float32),
        compiler_params=pltpu.CompilerParams(
            skip_device_barrier=True,
            disable_semaphore_checks=True,
        ),
        scratch_types=[
            pltpu.VMEM((_N_CHUNK, _CHUNK), jnp.int32),
            pltpu.VMEM((_N_CHUNK, _CHUNK), jnp.int32),
            pltpu.VMEM((_N_CHUNK, _CHUNK), jnp.int32),
            pltpu.VMEM((_P_PER_W, _PAIR_D), jnp.float32),
            pltpu.SemaphoreType.DMA,
        ],
    )
    def gather_kernel(idx_hbm, t2_hbm, out_hbm, ia_v, ib_v, pair_v, rows_v,
                      sem):
        sid = lax.axis_index("s")
        wid = sid * _NC + lax.axis_index("c")
        base = wid * _P_PER_W
        for j in range(_N_CHUNK):
            pltpu.sync_copy(idx_hbm.at[pl.ds(base + j * _CHUNK, _CHUNK)],
                            ia_v.at[j])
            pltpu.sync_copy(
                idx_hbm.at[pl.ds(_HALF + base + j * _CHUNK, _CHUNK)],
                ib_v.at[j])
        for j in range(_N_CHUNK):
            for v in range(_CHUNK // _L):
                a = ia_v[j, pl.ds(v * _L, _L)]
                b = ib_v[j, pl.ds(v * _L, _L)]
                pair_v[j, pl.ds(v * _L, _L)] = a * _ROWS + b
        copies = []
        for j in range(_N_CHUNK):
            copies.append(pltpu.async_copy(
                t2_hbm.at[pair_v.at[j]],
                rows_v.at[pl.ds(j * _CHUNK, _CHUNK)],
                sem,
            ))
        for c in copies:
            c.wait()
        pltpu.sync_copy(rows_v, out_hbm.at[pl.ds(base, _P_PER_W)])

    return gather_kernel


_gather = _make_gather()

_TC_BLK = 512


def _relayout_body(x_ref, o_ref):
    i = pl.program_id(0)
    x = x_ref[...]
    o_ref[...] = jnp.where(i < _HALF // _TC_BLK,
                           x[:, :EMBED_DIM], x[:, EMBED_DIM:])


_relayout = pl.pallas_call(
    _relayout_body,
    out_shape=jax.ShapeDtypeStruct((BATCH, EMBED_DIM), jnp.float32),
    grid=(BATCH // _TC_BLK,),
    in_specs=[pl.BlockSpec(
        (_TC_BLK, _PAIR_D),
        lambda i: (i % (_HALF // _TC_BLK), 0),
    )],
    out_specs=pl.BlockSpec((_TC_BLK, EMBED_DIM), lambda i: (i, 0)),
)


def kernel(biome_labels, table):
    idx = biome_labels.astype(jnp.int32)
    padded = jnp.pad(table, ((0, _ROWS - NUM_BIOMES), (0, 0)))
    t2 = jnp.concatenate(
        [jnp.repeat(padded, _ROWS, axis=0), jnp.tile(padded, (_ROWS, 1))],
        axis=1,
    )
    paired = _gather(idx, t2)
    return _relayout(paired)


# speedup vs baseline: 2.7407x; 1.5742x over previous
import functools

import jax
import jax.numpy as jnp
from jax import lax
from jax.experimental import pallas as pl
from jax.experimental.pallas import tpu as pltpu
from jax.experimental.pallas import tpu_sc as plsc

NUM_BIOMES = 11
EMBED_DIM = 64
BATCH = 16384

_info = plsc.get_sparse_core_info()
_NC, _NS = _info.num_cores, _info.num_subcores
_NW = _NC * _NS
_B_PER_W = BATCH // _NW
_CHUNK = 128
_N_CHUNK = _B_PER_W // _CHUNK


def _make_gather():
    mesh = plsc.VectorSubcoreMesh(core_axis_name="c", subcore_axis_name="s")

    @functools.partial(
        pl.kernel,
        mesh=mesh,
        out_type=jax.ShapeDtypeStruct((BATCH, EMBED_DIM), jnp.float32),
        compiler_params=pltpu.CompilerParams(
            use_tc_tiling_on_sc=False,
            skip_device_barrier=True,
            disable_semaphore_checks=True,
        ),
        scratch_types=[
            pltpu.VMEM((_N_CHUNK, _CHUNK), jnp.int32),
            pltpu.VMEM((_B_PER_W, EMBED_DIM), jnp.float32),
            pltpu.VMEM_SHARED((NUM_BIOMES, EMBED_DIM), jnp.float32),
            pltpu.SemaphoreType.DMA,
            pltpu.SemaphoreType.DMA,
            pltpu.SemaphoreType.DMA,
        ],
    )
    def gather_kernel(idx_hbm, table_hbm, out_hbm, idx_v, rows_v, table_sh,
                      sem_i, sem_g, sem_w):
        sid = lax.axis_index("s")
        wid = sid * _NC + lax.axis_index("c")
        base = wid * _B_PER_W
        @pl.when(sid == 0)
        def _():
            pltpu.sync_copy(table_hbm, table_sh)
        idx_cp = [
            pltpu.async_copy(idx_hbm.at[pl.ds(base + j * _CHUNK, _CHUNK)],
                             idx_v.at[j], sem_i)
            for j in range(_N_CHUNK)
        ]
        plsc.subcore_barrier()
        gathers = []
        for j in range(_N_CHUNK):
            idx_cp[j].wait()
            gathers.append(pltpu.async_copy(
                table_sh.at[idx_v.at[j]],
                rows_v.at[pl.ds(j * _CHUNK, _CHUNK)],
                sem_g,
            ))
        writes = []
        for j in range(_N_CHUNK):
            gathers[j].wait()
            writes.append(pltpu.async_copy(
                rows_v.at[pl.ds(j * _CHUNK, _CHUNK)],
                out_hbm.at[pl.ds(base + j * _CHUNK, _CHUNK)],
                sem_w,
            ))
        for w in writes:
            w.wait()

    return gather_kernel


_gather = _make_gather()


def kernel(biome_labels, table):
    idx = biome_labels.astype(jnp.int32)
    return _gather(idx, table)
